# baseline (speedup 1.0000x reference)
import functools

import jax
import jax.numpy as jnp
from jax import lax
from jax.experimental import pallas as pl
from jax.experimental.pallas import tpu as pltpu
from jax.experimental.pallas import tpu_sc as plsc

D = 64
B = 16384
H0 = 128
NC = 2
NS = 16
NW = NC * NS
BPW = B // NW
IDX_CH = 128
NCH = BPW // IDX_CH

_CONTRACT0 = (((0,), (0,)), ((), ()))


def _pmat_body(tT_ref, w_ref, out_ref):
    out_ref[...] = lax.dot_general(
        tT_ref[...], w_ref[...], _CONTRACT0,
        preferred_element_type=jnp.float32)


def _pmat_call(tT, w, n_rows, blk):
    return pl.pallas_call(
        _pmat_body,
        grid=(pl.cdiv(n_rows, blk),),
        in_specs=[
            pl.BlockSpec((D, blk), lambda i: (0, i)),
            pl.BlockSpec((D, H0), lambda i: (0, 0)),
        ],
        out_specs=pl.BlockSpec((blk, H0), lambda i: (i, 0)),
        out_shape=jax.ShapeDtypeStruct((n_rows, H0), jnp.float32),
    )(tT, w)


@functools.cache
def _make_sc_gather():
    mesh = plsc.VectorSubcoreMesh(core_axis_name="c", subcore_axis_name="s")

    @functools.partial(
        pl.kernel,
        mesh=mesh,
        out_type=[
            jax.ShapeDtypeStruct((B, H0), jnp.float32),
            jax.ShapeDtypeStruct((B, H0), jnp.float32),
        ],
        scratch_types=[
            pltpu.VMEM((NCH, IDX_CH), jnp.int32),
            pltpu.VMEM((BPW, H0), jnp.float32),
            pltpu.SemaphoreType.DMA,
        ],
    )
    def _sc_gather(user_hbm, item_hbm, pu_hbm, pi_hbm, xu_out, xi_out,
                   idx_v, slab_v, sem):
        wid = lax.axis_index("s") * NC + lax.axis_index("c")
        base = wid * BPW
        pltpu.sync_copy(user_hbm.at[wid], idx_v)
        for j in range(NCH):
            pltpu.async_copy(
                pu_hbm.at[idx_v.at[j]],
                slab_v.at[pl.ds(j * IDX_CH, IDX_CH)], sem)
        pltpu.make_async_copy(pu_hbm.at[pl.ds(0, BPW)], slab_v, sem).wait()
        pltpu.sync_copy(slab_v, xu_out.at[pl.ds(base, BPW)])

        pltpu.sync_copy(item_hbm.at[wid], idx_v)
        for j in range(NCH):
            pltpu.async_copy(
                pi_hbm.at[idx_v.at[j]],
                slab_v.at[pl.ds(j * IDX_CH, IDX_CH)], sem)
        pltpu.make_async_copy(pi_hbm.at[pl.ds(0, BPW)], slab_v, sem).wait()
        pltpu.sync_copy(slab_v, xi_out.at[pl.ds(base, BPW)])

    return _sc_gather


MLP_BLK = 2048


def _mlp_body(xu_ref, xi_ref, b0_ref, w1_ref, b1_ref,
              w2_ref, b2_ref, wo_ref, bo_ref, out_ref):
    x = jnp.maximum(xu_ref[...] + xi_ref[...] + b0_ref[...], 0.0)
    x = jnp.dot(x, w1_ref[...], preferred_element_type=jnp.float32)
    x = jnp.maximum(x + b1_ref[...], 0.0)
    x = jnp.dot(x, w2_ref[...], preferred_element_type=jnp.float32)
    x = jnp.maximum(x + b2_ref[...], 0.0)
    z = jnp.sum(x * wo_ref[...], axis=1) + bo_ref[0]
    out_ref[...] = 1.0 / (1.0 + jnp.exp(-z))


def _mlp_call(xu, xi, b0, w1, b1, w2, b2, wo, bo):
    n_blk = B // MLP_BLK
    full2d = lambda shape: pl.BlockSpec(shape, lambda i: (0, 0))
    return pl.pallas_call(
        _mlp_body,
        grid=(n_blk,),
        in_specs=[
            pl.BlockSpec((MLP_BLK, H0), lambda i: (i, 0)),
            pl.BlockSpec((MLP_BLK, H0), lambda i: (i, 0)),
            full2d((1, H0)),
            full2d((H0, 64)),
            full2d((1, 64)),
            full2d((64, 32)),
            full2d((1, 32)),
            full2d((1, 32)),
            pl.BlockSpec((1,), lambda i: (0,)),
        ],
        out_specs=pl.BlockSpec((MLP_BLK,), lambda i: (i,)),
        out_shape=jax.ShapeDtypeStruct((B,), jnp.float32),
    )(xu, xi, b0, w1, b1, w2, b2, wo, bo)


def kernel(user, item, user_table, item_table, W0, b0, W1, b1, W2, b2, Wo, bo):
    pu = _pmat_call(user_table.T, W0[:, :D].T, 1000000, 2048)
    pi = _pmat_call(item_table.T, W0[:, D:].T, 100000, 2048)
    user_r = user.reshape(NW, NCH, IDX_CH)
    item_r = item.reshape(NW, NCH, IDX_CH)
    xu, xi = _make_sc_gather()(user_r, item_r, pu, pi)
    return _mlp_call(
        xu, xi,
        b0.reshape(1, H0),
        W1.T, b1.reshape(1, 64),
        W2.T, b2.reshape(1, 32),
        Wo.reshape(1, 32), bo)


# speedup vs baseline: 2.1290x; 2.1290x over previous
import functools

import jax
import jax.numpy as jnp
from jax import lax
from jax.experimental import pallas as pl
from jax.experimental.pallas import tpu as pltpu
from jax.experimental.pallas import tpu_sc as plsc

D = 64
B = 16384
H0 = 128
NU = 1000000
NI = 100000
NC = 2
NS = 16
NW = NC * NS
BPW = B // NW
IDX_CH = 128
NCH = BPW // IDX_CH

PBLK = 8192
HU = 62 * PBLK
HI = 7 * PBLK
NBU_LAST = pl.cdiv(NU, PBLK) - 1
NBI_LAST = pl.cdiv(NI, PBLK) - 1

MASK_HI16 = -65536

_CONTRACT0 = (((0,), (0,)), ((), ()))


def _rne_bf16_bits(x):
    i = lax.bitcast_convert_type(x, jnp.int32)
    r = i + 0x7FFF + (lax.shift_right_logical(i, 16) & 1)
    return r & MASK_HI16


def _p2_body(lo_ref, hi_ref, w_ref, out_ref):
    plo = lax.dot_general(lo_ref[...], w_ref[...], _CONTRACT0,
                          preferred_element_type=jnp.float32)
    phi = lax.dot_general(hi_ref[...], w_ref[...], _CONTRACT0,
                          preferred_element_type=jnp.float32)
    blo = _rne_bf16_bits(plo)
    bhi = _rne_bf16_bits(phi)
    out_ref[...] = blo | lax.shift_right_logical(bhi, 16)


def _p2_call(tT, w, h_rows, nb_last):
    nbh = h_rows // PBLK

    return pl.pallas_call(
        _p2_body,
        grid=(nbh,),
        in_specs=[
            pl.BlockSpec((D, PBLK), lambda i: (0, i)),
            pl.BlockSpec((D, PBLK),
                         lambda i: (0, jnp.minimum(i + nbh, nb_last))),
            pl.BlockSpec((D, H0), lambda i: (0, 0)),
        ],
        out_specs=pl.BlockSpec((PBLK, H0), lambda i: (i, 0)),
        out_shape=jax.ShapeDtypeStruct((h_rows, H0), jnp.int32),
    )(tT, tT, w)


@functools.cache
def _make_sc_gather():
    mesh = plsc.VectorSubcoreMesh(core_axis_name="c", subcore_axis_name="s")

    @functools.partial(
        pl.kernel,
        mesh=mesh,
        out_type=[
            jax.ShapeDtypeStruct((B, H0), jnp.int32),
            jax.ShapeDtypeStruct((B, H0), jnp.int32),
        ],
        scratch_types=[
            pltpu.VMEM((NCH, IDX_CH), jnp.int32),
            pltpu.VMEM((BPW, H0), jnp.int32),
            pltpu.SemaphoreType.DMA,
        ],
    )
    def _sc_gather(user_hbm, item_hbm, pu_hbm, pi_hbm, xu_out, xi_out,
                   idx_v, slab_v, sem):
        wid = lax.axis_index("s") * NC + lax.axis_index("c")
        base = wid * BPW

        def remap(h):
            for j in range(NCH):
                for g in range(IDX_CH // 16):
                    v = idx_v[j, pl.ds(g * 16, 16)]
                    v = jnp.where(v >= h, v - h, v)
                    idx_v[j, pl.ds(g * 16, 16)] = v

        pltpu.sync_copy(user_hbm.at[wid], idx_v)
        remap(HU)
        for j in range(NCH):
            pltpu.async_copy(
                pu_hbm.at[idx_v.at[j]],
                slab_v.at[pl.ds(j * IDX_CH, IDX_CH)], sem)
        pltpu.make_async_copy(pu_hbm.at[pl.ds(0, BPW)], slab_v, sem).wait()
        pltpu.sync_copy(slab_v, xu_out.at[pl.ds(base, BPW)])

        pltpu.sync_copy(item_hbm.at[wid], idx_v)
        remap(HI)
        for j in range(NCH):
            pltpu.async_copy(
                pi_hbm.at[idx_v.at[j]],
                slab_v.at[pl.ds(j * IDX_CH, IDX_CH)], sem)
        pltpu.make_async_copy(pi_hbm.at[pl.ds(0, BPW)], slab_v, sem).wait()
        pltpu.sync_copy(slab_v, xi_out.at[pl.ds(base, BPW)])

    return _sc_gather


MLP_BLK = 2048


def _unpack_half(w, sel_hi):
    bits = jnp.where(sel_hi, lax.shift_left(w, 16), w & MASK_HI16)
    return lax.bitcast_convert_type(bits, jnp.float32)


def _mlp_body(xu_ref, xi_ref, u_ref, i_ref, b0_ref, w1_ref, b1_ref,
              w2_ref, b2_ref, wo_ref, bo_ref, out_ref):
    usel = u_ref[...] >= HU
    isel = i_ref[...] >= HI
    xu = _unpack_half(xu_ref[...], usel)
    xi = _unpack_half(xi_ref[...], isel)
    x = jnp.maximum(xu + xi + b0_ref[...], 0.0)
    x = jnp.dot(x, w1_ref[...], preferred_element_type=jnp.float32)
    x = jnp.maximum(x + b1_ref[...], 0.0)
    x = jnp.dot(x, w2_ref[...], preferred_element_type=jnp.float32)
    x = jnp.maximum(x + b2_ref[...], 0.0)
    z = jnp.sum(x * wo_ref[...], axis=1) + bo_ref[0]
    out_ref[...] = 1.0 / (1.0 + jnp.exp(-z))


def _mlp_call(xu, xi, user, item, b0, w1, b1, w2, b2, wo, bo):
    n_blk = B // MLP_BLK
    full2d = lambda shape: pl.BlockSpec(shape, lambda i: (0, 0))
    return pl.pallas_call(
        _mlp_body,
        grid=(n_blk,),
        in_specs=[
            pl.BlockSpec((MLP_BLK, H0), lambda i: (i, 0)),
            pl.BlockSpec((MLP_BLK, H0), lambda i: (i, 0)),
            pl.BlockSpec((MLP_BLK, 1), lambda i: (i, 0)),
            pl.BlockSpec((MLP_BLK, 1), lambda i: (i, 0)),
            full2d((1, H0)),
            full2d((H0, 64)),
            full2d((1, 64)),
            full2d((64, 32)),
            full2d((1, 32)),
            full2d((1, 32)),
            pl.BlockSpec((1,), lambda i: (0,)),
        ],
        out_specs=pl.BlockSpec((MLP_BLK,), lambda i: (i,)),
        out_shape=jax.ShapeDtypeStruct((B,), jnp.float32),
    )(xu, xi, user, item, b0, w1, b1, w2, b2, wo, bo)


def kernel(user, item, user_table, item_table, W0, b0, W1, b1, W2, b2, Wo, bo):
    pu = _p2_call(user_table.T, W0[:, :D].T, HU, NBU_LAST)
    pi = _p2_call(item_table.T, W0[:, D:].T, HI, NBI_LAST)
    user_r = user.reshape(NW, NCH, IDX_CH)
    item_r = item.reshape(NW, NCH, IDX_CH)
    xu, xi = _make_sc_gather()(user_r, item_r, pu, pi)
    return _mlp_call(
        xu, xi, user.reshape(B, 1), item.reshape(B, 1),
        b0.reshape(1, H0),
        W1.T, b1.reshape(1, 64),
        W2.T, b2.reshape(1, 32),
        Wo.reshape(1, 32), bo)


# speedup vs baseline: 2.2958x; 1.0784x over previous
import functools

import jax
import jax.numpy as jnp
from jax import lax
from jax.experimental import pallas as pl
from jax.experimental.pallas import tpu as pltpu
from jax.experimental.pallas import tpu_sc as plsc

D = 64
B = 16384
H0 = 128
NU = 1000000
NI = 100000
NC = 2
NS = 16
NW = NC * NS
BPW = B // NW
IDX_CH = 128
NCH = BPW // IDX_CH

PBLK = 8192
HU = 62 * PBLK
HI = 7 * PBLK
NBU_LAST = pl.cdiv(NU, PBLK) - 1
NBI_LAST = pl.cdiv(NI, PBLK) - 1

MASK_HI16 = -65536

_CONTRACT0 = (((0,), (0,)), ((), ()))


def _rne_bf16_bits(x):
    i = lax.bitcast_convert_type(x, jnp.int32)
    r = i + 0x7FFF + (lax.shift_right_logical(i, 16) & 1)
    return r & MASK_HI16


def _p2_body(lo_ref, hi_ref, w_ref, out_ref):
    plo = lax.dot_general(lo_ref[...], w_ref[...], _CONTRACT0,
                          preferred_element_type=jnp.float32)
    phi = lax.dot_general(hi_ref[...], w_ref[...], _CONTRACT0,
                          preferred_element_type=jnp.float32)
    blo = _rne_bf16_bits(plo)
    bhi = _rne_bf16_bits(phi)
    out_ref[...] = blo | lax.shift_right_logical(bhi, 16)


def _p2_call(tT, w, h_rows, n_rows, blk):
    nbh = h_rows // blk
    nb_last = pl.cdiv(n_rows, blk) - 1

    return pl.pallas_call(
        _p2_body,
        grid=(nbh,),
        in_specs=[
            pl.BlockSpec((D, blk), lambda i: (0, i)),
            pl.BlockSpec((D, blk),
                         lambda i: (0, jnp.minimum(i + nbh, nb_last))),
            pl.BlockSpec((D, H0), lambda i: (0, 0)),
        ],
        out_specs=pl.BlockSpec((blk, H0), lambda i: (i, 0)),
        out_shape=jax.ShapeDtypeStruct((h_rows, H0), jnp.int32),
    )(tT, tT, w)


@functools.cache
def _make_sc_gather():
    mesh = plsc.VectorSubcoreMesh(core_axis_name="c", subcore_axis_name="s")

    @functools.partial(
        pl.kernel,
        mesh=mesh,
        out_type=[
            jax.ShapeDtypeStruct((B, H0), jnp.int32),
            jax.ShapeDtypeStruct((B, H0), jnp.int32),
        ],
        scratch_types=[
            pltpu.VMEM((NCH, IDX_CH), jnp.int32),
            pltpu.VMEM((BPW, H0), jnp.int32),
            pltpu.SemaphoreType.DMA,
        ],
    )
    def _sc_gather(user_hbm, item_hbm, pu_hbm, pi_hbm, xu_out, xi_out,
                   idx_v, slab_v, sem):
        wid = lax.axis_index("s") * NC + lax.axis_index("c")
        base = wid * BPW

        def remap(h):
            for j in range(NCH):
                for g in range(IDX_CH // 16):
                    v = idx_v[j, pl.ds(g * 16, 16)]
                    v = jnp.where(v >= h, v - h, v)
                    idx_v[j, pl.ds(g * 16, 16)] = v

        pltpu.sync_copy(user_hbm.at[wid], idx_v)
        remap(HU)
        for j in range(NCH):
            pltpu.async_copy(
                pu_hbm.at[idx_v.at[j]],
                slab_v.at[pl.ds(j * IDX_CH, IDX_CH)], sem)
        pltpu.make_async_copy(pu_hbm.at[pl.ds(0, BPW)], slab_v, sem).wait()
        pltpu.sync_copy(slab_v, xu_out.at[pl.ds(base, BPW)])

        pltpu.sync_copy(item_hbm.at[wid], idx_v)
        remap(HI)
        for j in range(NCH):
            pltpu.async_copy(
                pi_hbm.at[idx_v.at[j]],
                slab_v.at[pl.ds(j * IDX_CH, IDX_CH)], sem)
        pltpu.make_async_copy(pi_hbm.at[pl.ds(0, BPW)], slab_v, sem).wait()
        pltpu.sync_copy(slab_v, xi_out.at[pl.ds(base, BPW)])

    return _sc_gather


MLP_BLK = 2048


def _unpack_half(w, sel_hi):
    bits = jnp.where(sel_hi, lax.shift_left(w, 16), w & MASK_HI16)
    return lax.bitcast_convert_type(bits, jnp.float32)


def _mlp_body(xu_ref, xi_ref, u_ref, i_ref, b0_ref, w1_ref, b1_ref,
              w2_ref, b2_ref, wo_ref, bo_ref, out_ref):
    usel = u_ref[...] >= HU
    isel = i_ref[...] >= HI
    xu = _unpack_half(xu_ref[...], usel)
    xi = _unpack_half(xi_ref[...], isel)
    x = jnp.maximum(xu + xi + b0_ref[...], 0.0)
    x = jnp.dot(x, w1_ref[...], preferred_element_type=jnp.float32)
    x = jnp.maximum(x + b1_ref[...], 0.0)
    x = jnp.dot(x, w2_ref[...], preferred_element_type=jnp.float32)
    x = jnp.maximum(x + b2_ref[...], 0.0)
    z = jnp.sum(x * wo_ref[...], axis=1) + bo_ref[0]
    out_ref[...] = 1.0 / (1.0 + jnp.exp(-z))


def _mlp_call(xu, xi, user, item, b0, w1, b1, w2, b2, wo, bo):
    n_blk = B // MLP_BLK
    full2d = lambda shape: pl.BlockSpec(shape, lambda i: (0, 0))
    return pl.pallas_call(
        _mlp_body,
        grid=(n_blk,),
        in_specs=[
            pl.BlockSpec((MLP_BLK, H0), lambda i: (i, 0)),
            pl.BlockSpec((MLP_BLK, H0), lambda i: (i, 0)),
            pl.BlockSpec((MLP_BLK, 1), lambda i: (i, 0)),
            pl.BlockSpec((MLP_BLK, 1), lambda i: (i, 0)),
            full2d((1, H0)),
            full2d((H0, 64)),
            full2d((1, 64)),
            full2d((64, 32)),
            full2d((1, 32)),
            full2d((1, 32)),
            pl.BlockSpec((1,), lambda i: (0,)),
        ],
        out_specs=pl.BlockSpec((MLP_BLK,), lambda i: (i,)),
        out_shape=jax.ShapeDtypeStruct((B,), jnp.float32),
    )(xu, xi, user, item, b0, w1, b1, w2, b2, wo, bo)


def kernel(user, item, user_table, item_table, W0, b0, W1, b1, W2, b2, Wo, bo):
    pu = _p2_call(user_table.T, W0[:, :D].T, HU, NU, 16384)
    pi = _p2_call(item_table.T, W0[:, D:].T, HI, NI, 8192)
    user_r = user.reshape(NW, NCH, IDX_CH)
    item_r = item.reshape(NW, NCH, IDX_CH)
    xu, xi = _make_sc_gather()(user_r, item_r, pu, pi)
    return _mlp_call(
        xu, xi, user.reshape(B, 1), item.reshape(B, 1),
        b0.reshape(1, H0),
        W1.T, b1.reshape(1, 64),
        W2.T, b2.reshape(1, 32),
        Wo.reshape(1, 32), bo)
